# baseline (speedup 1.0000x reference)
import functools

import jax
import jax.numpy as jnp
from jax import lax
from jax.experimental import pallas as pl
from jax.experimental.pallas import tpu as pltpu
from jax.experimental.pallas import tpu_sc as plsc

_N = 10000
_E = 320000
_D = 128
_EF = 16
_L = 3
_S = 10000

_NC = 2
_NS = 16
_NW = _NC * _NS
_EW = _E // _NW
_K = 40
_NB = _EW // _K
_RPS = 624


_BE = 2000
_GE = _E // _BE


def _edge_mlp_body(ea_ref, w1_ref, b1_ref, w2_ref, b2_ref, out_ref):
    t = jnp.dot(ea_ref[...], w1_ref[0], preferred_element_type=jnp.float32)
    t = jnp.maximum(t + b1_ref[0], 0.0)
    out_ref[...] = jnp.dot(t, w2_ref[0], preferred_element_type=jnp.float32) + b2_ref[0]


def _edge_mlp(edge_attr, be1_W, be1_b, be2_W, be2_b):
    return pl.pallas_call(
        _edge_mlp_body,
        grid=(_L * _GE,),
        in_specs=[
            pl.BlockSpec((_BE, _EF), lambda i: (i % _GE, 0)),
            pl.BlockSpec((1, _EF, _D), lambda i: (i // _GE, 0, 0)),
            pl.BlockSpec((1, 1, _D), lambda i: (i // _GE, 0, 0)),
            pl.BlockSpec((1, _D, _D), lambda i: (i // _GE, 0, 0)),
            pl.BlockSpec((1, 1, _D), lambda i: (i // _GE, 0, 0)),
        ],
        out_specs=pl.BlockSpec((_BE, _D), lambda i: (i, 0)),
        out_shape=jax.ShapeDtypeStruct((_L * _E, _D), jnp.float32),
    )(edge_attr, be1_W, be1_b.reshape(_L, 1, _D), be2_W, be2_b.reshape(_L, 1, _D))


def _node_body(h_ref, p_ref, w1_ref, b1_ref, w2_ref, b2_ref, g_ref, bb_ref,
               eps_ref, mask_ref, out_ref, *, residual, final_mask):
    h = h_ref[...]
    z = (1.0 + eps_ref[0, 0]) * h + p_ref[0] + p_ref[1]
    z = jnp.maximum(
        jnp.dot(z, w1_ref[...], preferred_element_type=jnp.float32) + b1_ref[...], 0.0)
    z = jnp.dot(z, w2_ref[...], preferred_element_type=jnp.float32) + b2_ref[...]
    mu = jnp.mean(z, axis=0, keepdims=True)
    var = jnp.mean((z - mu) ** 2, axis=0, keepdims=True)
    z = (z - mu) / jnp.sqrt(var + 1e-5) * g_ref[...] + bb_ref[...]
    z = jnp.maximum(z, 0.0)
    if residual:
        z = h + z
    if final_mask:
        z = z * mask_ref[...]
    out_ref[...] = z


def _node_update(h, parts, w1, b1, w2, b2, g, bb, eps_l, mask, residual, final_mask):
    body = functools.partial(_node_body, residual=residual, final_mask=final_mask)
    return pl.pallas_call(
        body,
        in_specs=[
            pl.BlockSpec(memory_space=pltpu.VMEM),
            pl.BlockSpec(memory_space=pltpu.VMEM),
            pl.BlockSpec(memory_space=pltpu.VMEM),
            pl.BlockSpec(memory_space=pltpu.VMEM),
            pl.BlockSpec(memory_space=pltpu.VMEM),
            pl.BlockSpec(memory_space=pltpu.VMEM),
            pl.BlockSpec(memory_space=pltpu.VMEM),
            pl.BlockSpec(memory_space=pltpu.VMEM),
            pl.BlockSpec(memory_space=pltpu.SMEM),
            pl.BlockSpec(memory_space=pltpu.VMEM),
        ],
        out_shape=jax.ShapeDtypeStruct((_N, _D), jnp.float32),
    )(h, parts, w1, b1, w2, b2, g, bb, eps_l, mask)


def _sc_layer_body(h_hbm, ee_hbm, src_hbm, dst_hbm, ew_hbm, out_hbm,
                   srcv, dstv, wv, ebuf, gbuf, zbuf, acc,
                   lsem, gsem, ssem, dsem, wsem, *, layer):
    c = lax.axis_index("c")
    s = lax.axis_index("s")
    wid = c * _NS + s

    def zrow(i, _):
        for j in range(_D // 16):
            zbuf[i, pl.ds(j * 16, 16)] = jnp.zeros((16,), jnp.float32)
        return 0
    lax.fori_loop(0, 48, zrow, 0)
    for k in range(_RPS // 48):
        off = pl.multiple_of(s * _RPS + k * 48, 8)
        pltpu.sync_copy(zbuf, acc.at[pl.ds(off, 48)])

    @pl.when(s == _NS - 1)
    def _ztail():
        pltpu.sync_copy(zbuf.at[pl.ds(0, 16)], acc.at[pl.ds(_NS * _RPS, 16)])

    ebase0 = pl.multiple_of(wid * _EW, 8)
    pltpu.sync_copy(src_hbm.at[pl.ds(ebase0, _EW)], srcv)
    plsc.subcore_barrier()

    ebase = layer * _E + wid * _EW

    def issue_meta(b):
        base0 = pl.multiple_of(wid * _EW + b * _K, 8)
        p8 = lax.rem(b, 8)
        p4 = lax.rem(b, 4)
        pltpu.async_copy(dst_hbm.at[pl.ds(base0, _K)], dstv.at[p8],
                         dsem.at[p8])
        pltpu.async_copy(ew_hbm.at[pl.ds(base0, _K)],
                         wv.at[p4].at[pl.ds(0, _K)], wsem.at[p4])

    def wait_meta(b):
        base0 = pl.multiple_of(wid * _EW + b * _K, 8)
        p8 = lax.rem(b, 8)
        p4 = lax.rem(b, 4)
        pltpu.make_async_copy(dst_hbm.at[pl.ds(base0, _K)], dstv.at[p8],
                              dsem.at[p8]).wait()
        pltpu.make_async_copy(ew_hbm.at[pl.ds(base0, _K)],
                              wv.at[p4].at[pl.ds(0, _K)], wsem.at[p4]).wait()

    def issue_in(b):
        p4 = lax.rem(b, 4)
        p2 = lax.rem(b, 2)
        base1 = pl.multiple_of(ebase + b * _K, 8)
        pltpu.async_copy(ee_hbm.at[pl.ds(base1, _K)], ebuf.at[p4], lsem.at[p4])
        pltpu.async_copy(h_hbm.at[srcv.at[pl.ds(b * _K, _K)]], gbuf.at[p2],
                         gsem.at[p2])

    def wait_in(b):
        p4 = lax.rem(b, 4)
        p2 = lax.rem(b, 2)
        pltpu.make_async_copy(ee_hbm.at[pl.ds(ebase, _K)], ebuf.at[p4],
                              lsem.at[p4]).wait()
        pltpu.make_async_copy(h_hbm.at[srcv.at[pl.ds(b * _K, _K)]],
                              gbuf.at[p2], gsem.at[p2]).wait()

    def wait_scatter(b):
        p4 = lax.rem(b, 4)
        p8 = lax.rem(b, 8)
        pltpu.make_async_copy(ebuf.at[p4], acc.at[dstv.at[p8]],
                              ssem.at[p4]).wait()

    for b in range(3):
        issue_meta(b)
    issue_in(0)

    def block(b, _):
        p4 = lax.rem(b, 4)
        p2 = lax.rem(b, 2)

        @pl.when(b >= 3)
        def _():
            wait_scatter(b - 3)

        @pl.when(b + 3 < _NB)
        def _():
            issue_meta(b + 3)

        @pl.when(b + 1 < _NB)
        def _():
            issue_in(b + 1)

        wait_in(b)
        wait_meta(b)

        def edge(i, _):
            w = wv[p4, pl.ds(i, 16)][0]
            for j in range(_D // 16):
                sl = pl.ds(j * 16, 16)
                ebuf[p4, i, sl] = jnp.maximum(ebuf[p4, i, sl] + gbuf[p2, i, sl],
                                              0.0) * w
            return 0
        lax.fori_loop(0, _K, edge, 0)

        pltpu.async_copy(ebuf.at[p4], acc.at[dstv.at[lax.rem(b, 8)]],
                         ssem.at[p4], add=True)
        return 0
    lax.fori_loop(0, _NB, block, 0)

    for b in (_NB - 3, _NB - 2, _NB - 1):
        wait_scatter(b)

    plsc.subcore_barrier()
    off = pl.multiple_of(s * _RPS, 8)
    pltpu.sync_copy(acc.at[pl.ds(off, _RPS)],
                    out_hbm.at[c].at[pl.ds(off, _RPS)])

    @pl.when(s == _NS - 1)
    def _otail():
        pltpu.sync_copy(acc.at[pl.ds(_NS * _RPS, 16)],
                        out_hbm.at[c].at[pl.ds(_NS * _RPS, 16)])


def _sc_layer(h, eemb, src, dst2, ew, layer):
    body = functools.partial(_sc_layer_body, layer=layer)
    mesh = plsc.VectorSubcoreMesh(core_axis_name="c", subcore_axis_name="s")
    f = pl.kernel(
        body,
        out_type=jax.ShapeDtypeStruct((_NC, _N, _D), jnp.float32),
        mesh=mesh,
        scratch_types=[
            pltpu.VMEM((_EW,), jnp.int32),
            pltpu.VMEM((8, _K), jnp.int32),
            pltpu.VMEM((4, _K + 16), jnp.float32),
            pltpu.VMEM((4, _K, _D), jnp.float32),
            pltpu.VMEM((2, _K, _D), jnp.float32),
            pltpu.VMEM((48, _D), jnp.float32),
            pltpu.VMEM_SHARED((_N, _D), jnp.float32),
            pltpu.SemaphoreType.DMA((4,)),
            pltpu.SemaphoreType.DMA((2,)),
            pltpu.SemaphoreType.DMA((4,)),
            pltpu.SemaphoreType.DMA((8,)),
            pltpu.SemaphoreType.DMA((4,)),
        ],
    )
    return f(h, eemb, src, dst2, ew)


_PK = 80
_PNB = _N // _PK


def _pool_body(hm_hbm, seg_hbm, out_hbm, idxv, buf, zbuf, acc):
    c = lax.axis_index("c")
    s = lax.axis_index("s")

    @pl.when(c == 0)
    def _():
        def zrow(i, _):
            for j in range(_D // 16):
                zbuf[i, pl.ds(j * 16, 16)] = jnp.zeros((16,), jnp.float32)
            return 0
        lax.fori_loop(0, 104, zrow, 0)
        for k in range(_RPS // 104):
            off = pl.multiple_of(s * _RPS + k * 104, 8)
            pltpu.sync_copy(zbuf, acc.at[pl.ds(off, 104)])

        @pl.when(s == _NS - 1)
        def _ztail():
            pltpu.sync_copy(zbuf.at[pl.ds(0, 16)], acc.at[pl.ds(_NS * _RPS, 16)])
        plsc.subcore_barrier()

        def block(t, _):
            blk = s + t * _NS
            @pl.when(blk < _PNB)
            def _():
                base = pl.multiple_of(blk * _PK, 8)
                pltpu.sync_copy(seg_hbm.at[pl.ds(base, _PK)], idxv.at[0])
                pltpu.sync_copy(hm_hbm.at[pl.ds(base, _PK)], buf)
                pltpu.sync_copy(buf, acc.at[idxv.at[0]], add=True)
            return 0
        lax.fori_loop(0, (_PNB + _NS - 1) // _NS, block, 0)

        plsc.subcore_barrier()
        off = pl.multiple_of(s * _RPS, 8)
        pltpu.sync_copy(acc.at[pl.ds(off, _RPS)], out_hbm.at[pl.ds(off, _RPS)])

        @pl.when(s == _NS - 1)
        def _otail():
            pltpu.sync_copy(acc.at[pl.ds(_NS * _RPS, 16)],
                            out_hbm.at[pl.ds(_NS * _RPS, 16)])


def _pool(hm, seg):
    mesh = plsc.VectorSubcoreMesh(core_axis_name="c", subcore_axis_name="s")
    f = pl.kernel(
        _pool_body,
        out_type=jax.ShapeDtypeStruct((_S, _D), jnp.float32),
        mesh=mesh,
        scratch_types=[
            pltpu.VMEM((1, _PK), jnp.int32),
            pltpu.VMEM((_PK, _D), jnp.float32),
            pltpu.VMEM((104, _D), jnp.float32),
            pltpu.VMEM_SHARED((_S, _D), jnp.float32),
        ],
    )
    return f(hm, seg)


def kernel(x, edge_index, edge_attr, edge_weight, node_mask, subgraphs2nodes,
           be1_W, be1_b, be2_W, be2_b, nn1_W, nn1_b, nn2_W, nn2_b,
           bn_g, bn_b, eps):
    src = edge_index[0]
    dst2 = edge_index[1]
    mask2d = node_mask.reshape(_N, 1)

    eemb = _edge_mlp(edge_attr, be1_W, be1_b, be2_W, be2_b)

    h = x
    for l in range(_L):
        parts = _sc_layer(h, eemb, src, dst2, edge_weight, l)
        h = _node_update(h, parts,
                         nn1_W[l], nn1_b[l:l + 1], nn2_W[l], nn2_b[l:l + 1],
                         bn_g[l:l + 1], bn_b[l:l + 1],
                         eps[l].reshape(1, 1), mask2d,
                         residual=(l > 0), final_mask=(l == _L - 1))

    return _pool(h, subgraphs2nodes)


# speedup vs baseline: 1.9060x; 1.9060x over previous
import functools

import jax
import jax.numpy as jnp
from jax import lax
from jax.experimental import pallas as pl
from jax.experimental.pallas import tpu as pltpu
from jax.experimental.pallas import tpu_sc as plsc

_N = 10000
_E = 320000
_D = 128
_EF = 16
_L = 3
_S = 10000

_NC = 2
_NS = 16
_NW = _NC * _NS
_EW = _E // _NW
_K = 40
_NB = _EW // _K
_RPS = 624


_BE = 2000
_GE = _E // _BE


def _edge_mlp_body(ea_ref, w1_ref, b1_ref, w2_ref, b2_ref, out_ref):
    t = jnp.dot(ea_ref[...], w1_ref[0], preferred_element_type=jnp.float32)
    t = jnp.maximum(t + b1_ref[0], 0.0)
    out_ref[...] = jnp.dot(t, w2_ref[0], preferred_element_type=jnp.float32) + b2_ref[0]


def _edge_mlp(edge_attr, be1_W, be1_b, be2_W, be2_b):
    return pl.pallas_call(
        _edge_mlp_body,
        grid=(_L * _GE,),
        in_specs=[
            pl.BlockSpec((_BE, _EF), lambda i: (i % _GE, 0)),
            pl.BlockSpec((1, _EF, _D), lambda i: (i // _GE, 0, 0)),
            pl.BlockSpec((1, 1, _D), lambda i: (i // _GE, 0, 0)),
            pl.BlockSpec((1, _D, _D), lambda i: (i // _GE, 0, 0)),
            pl.BlockSpec((1, 1, _D), lambda i: (i // _GE, 0, 0)),
        ],
        out_specs=pl.BlockSpec((_BE, _D), lambda i: (i, 0)),
        out_shape=jax.ShapeDtypeStruct((_L * _E, _D), jnp.float32),
    )(edge_attr, be1_W, be1_b.reshape(_L, 1, _D), be2_W, be2_b.reshape(_L, 1, _D))


def _node_body(h_ref, p_ref, w1_ref, b1_ref, w2_ref, b2_ref, g_ref, bb_ref,
               eps_ref, mask_ref, out_ref, *, residual, final_mask):
    h = h_ref[...]
    z = (1.0 + eps_ref[0, 0]) * h + p_ref[0] + p_ref[1]
    z = jnp.maximum(
        jnp.dot(z, w1_ref[...], preferred_element_type=jnp.float32) + b1_ref[...], 0.0)
    z = jnp.dot(z, w2_ref[...], preferred_element_type=jnp.float32) + b2_ref[...]
    mu = jnp.mean(z, axis=0, keepdims=True)
    var = jnp.mean((z - mu) ** 2, axis=0, keepdims=True)
    z = (z - mu) / jnp.sqrt(var + 1e-5) * g_ref[...] + bb_ref[...]
    z = jnp.maximum(z, 0.0)
    if residual:
        z = h + z
    if final_mask:
        z = z * mask_ref[...]
    out_ref[...] = z


def _node_update(h, parts, w1, b1, w2, b2, g, bb, eps_l, mask, residual, final_mask):
    body = functools.partial(_node_body, residual=residual, final_mask=final_mask)
    return pl.pallas_call(
        body,
        in_specs=[
            pl.BlockSpec(memory_space=pltpu.VMEM),
            pl.BlockSpec(memory_space=pltpu.VMEM),
            pl.BlockSpec(memory_space=pltpu.VMEM),
            pl.BlockSpec(memory_space=pltpu.VMEM),
            pl.BlockSpec(memory_space=pltpu.VMEM),
            pl.BlockSpec(memory_space=pltpu.VMEM),
            pl.BlockSpec(memory_space=pltpu.VMEM),
            pl.BlockSpec(memory_space=pltpu.VMEM),
            pl.BlockSpec(memory_space=pltpu.SMEM),
            pl.BlockSpec(memory_space=pltpu.VMEM),
        ],
        out_shape=jax.ShapeDtypeStruct((_N, _D), jnp.float32),
    )(h, parts, w1, b1, w2, b2, g, bb, eps_l, mask)


def _sc_layer_body(h_hbm, ee_hbm, src_hbm, dst_hbm, ew_hbm, out_hbm,
                   srcv, dstv, wv, ebuf, gbuf, zbuf, acc,
                   lsem, gsem, ssem, dsem, wsem, *, layer):
    c = lax.axis_index("c")
    s = lax.axis_index("s")
    wid = c * _NS + s

    def zrow(i, _):
        for j in range(_D // 16):
            zbuf[i, pl.ds(j * 16, 16)] = jnp.zeros((16,), jnp.float32)
        return 0
    lax.fori_loop(0, 48, zrow, 0)
    for k in range(_RPS // 48):
        off = pl.multiple_of(s * _RPS + k * 48, 8)
        pltpu.sync_copy(zbuf, acc.at[pl.ds(off, 48)])

    @pl.when(s == _NS - 1)
    def _ztail():
        pltpu.sync_copy(zbuf.at[pl.ds(0, 16)], acc.at[pl.ds(_NS * _RPS, 16)])

    ebase0 = pl.multiple_of(wid * _EW, 8)
    pltpu.sync_copy(src_hbm.at[pl.ds(ebase0, _EW)], srcv)
    plsc.subcore_barrier()

    ebase = layer * _E + wid * _EW

    def issue_meta(b):
        base0 = pl.multiple_of(wid * _EW + b * _K, 8)
        p8 = lax.rem(b, 8)
        p4 = lax.rem(b, 4)
        pltpu.async_copy(dst_hbm.at[pl.ds(base0, _K)], dstv.at[p8],
                         dsem.at[p8])
        pltpu.async_copy(ew_hbm.at[pl.ds(base0, _K)],
                         wv.at[p4].at[pl.ds(0, _K)], wsem.at[p4])

    def wait_meta(b):
        base0 = pl.multiple_of(wid * _EW + b * _K, 8)
        p8 = lax.rem(b, 8)
        p4 = lax.rem(b, 4)
        pltpu.make_async_copy(dst_hbm.at[pl.ds(base0, _K)], dstv.at[p8],
                              dsem.at[p8]).wait()
        pltpu.make_async_copy(ew_hbm.at[pl.ds(base0, _K)],
                              wv.at[p4].at[pl.ds(0, _K)], wsem.at[p4]).wait()

    def issue_in(b):
        p4 = lax.rem(b, 4)
        p2 = lax.rem(b, 2)
        base1 = pl.multiple_of(ebase + b * _K, 8)
        pltpu.async_copy(ee_hbm.at[pl.ds(base1, _K)], ebuf.at[p4], lsem.at[p4])
        pltpu.async_copy(h_hbm.at[srcv.at[pl.ds(b * _K, _K)]], gbuf.at[p2],
                         gsem.at[p2])

    def wait_in(b):
        p4 = lax.rem(b, 4)
        p2 = lax.rem(b, 2)
        pltpu.make_async_copy(ee_hbm.at[pl.ds(ebase, _K)], ebuf.at[p4],
                              lsem.at[p4]).wait()
        pltpu.make_async_copy(h_hbm.at[srcv.at[pl.ds(b * _K, _K)]],
                              gbuf.at[p2], gsem.at[p2]).wait()

    def wait_scatter(b):
        p4 = lax.rem(b, 4)
        p8 = lax.rem(b, 8)
        pltpu.make_async_copy(ebuf.at[p4], acc.at[dstv.at[p8]],
                              ssem.at[p4]).wait()

    for b in range(3):
        issue_meta(b)
    issue_in(0)

    def block(b, _):
        p4 = lax.rem(b, 4)
        p2 = lax.rem(b, 2)

        @pl.when(b >= 3)
        def _():
            wait_scatter(b - 3)

        @pl.when(b + 3 < _NB)
        def _():
            issue_meta(b + 3)

        @pl.when(b + 1 < _NB)
        def _():
            issue_in(b + 1)

        wait_in(b)
        wait_meta(b)

        @plsc.parallel_loop(0, _K, 1, unroll=4)
        def edge(i):
            w = wv[p4, pl.ds(i, 16)][0]
            for j in range(_D // 16):
                sl = pl.ds(j * 16, 16)
                ebuf[p4, i, sl] = jnp.maximum(ebuf[p4, i, sl] + gbuf[p2, i, sl],
                                              0.0) * w

        pltpu.async_copy(ebuf.at[p4], acc.at[dstv.at[lax.rem(b, 8)]],
                         ssem.at[p4], add=True)
        return 0
    lax.fori_loop(0, _NB, block, 0)

    for b in (_NB - 3, _NB - 2, _NB - 1):
        wait_scatter(b)

    plsc.subcore_barrier()
    off = pl.multiple_of(s * _RPS, 8)
    pltpu.sync_copy(acc.at[pl.ds(off, _RPS)],
                    out_hbm.at[c].at[pl.ds(off, _RPS)])

    @pl.when(s == _NS - 1)
    def _otail():
        pltpu.sync_copy(acc.at[pl.ds(_NS * _RPS, 16)],
                        out_hbm.at[c].at[pl.ds(_NS * _RPS, 16)])


def _sc_layer(h, eemb, src, dst2, ew, layer):
    body = functools.partial(_sc_layer_body, layer=layer)
    mesh = plsc.VectorSubcoreMesh(core_axis_name="c", subcore_axis_name="s")
    f = pl.kernel(
        body,
        out_type=jax.ShapeDtypeStruct((_NC, _N, _D), jnp.float32),
        mesh=mesh,
        scratch_types=[
            pltpu.VMEM((_EW,), jnp.int32),
            pltpu.VMEM((8, _K), jnp.int32),
            pltpu.VMEM((4, _K + 16), jnp.float32),
            pltpu.VMEM((4, _K, _D), jnp.float32),
            pltpu.VMEM((2, _K, _D), jnp.float32),
            pltpu.VMEM((48, _D), jnp.float32),
            pltpu.VMEM_SHARED((_N, _D), jnp.float32),
            pltpu.SemaphoreType.DMA((4,)),
            pltpu.SemaphoreType.DMA((2,)),
            pltpu.SemaphoreType.DMA((4,)),
            pltpu.SemaphoreType.DMA((8,)),
            pltpu.SemaphoreType.DMA((4,)),
        ],
    )
    return f(h, eemb, src, dst2, ew)


_PK = 80
_PNB = _N // _PK


def _pool_body(hm_hbm, seg_hbm, out_hbm, idxv, buf, zbuf, acc):
    c = lax.axis_index("c")
    s = lax.axis_index("s")

    @pl.when(c == 0)
    def _():
        def zrow(i, _):
            for j in range(_D // 16):
                zbuf[i, pl.ds(j * 16, 16)] = jnp.zeros((16,), jnp.float32)
            return 0
        lax.fori_loop(0, 104, zrow, 0)
        for k in range(_RPS // 104):
            off = pl.multiple_of(s * _RPS + k * 104, 8)
            pltpu.sync_copy(zbuf, acc.at[pl.ds(off, 104)])

        @pl.when(s == _NS - 1)
        def _ztail():
            pltpu.sync_copy(zbuf.at[pl.ds(0, 16)], acc.at[pl.ds(_NS * _RPS, 16)])
        plsc.subcore_barrier()

        def block(t, _):
            blk = s + t * _NS
            @pl.when(blk < _PNB)
            def _():
                base = pl.multiple_of(blk * _PK, 8)
                pltpu.sync_copy(seg_hbm.at[pl.ds(base, _PK)], idxv.at[0])
                pltpu.sync_copy(hm_hbm.at[pl.ds(base, _PK)], buf)
                pltpu.sync_copy(buf, acc.at[idxv.at[0]], add=True)
            return 0
        lax.fori_loop(0, (_PNB + _NS - 1) // _NS, block, 0)

        plsc.subcore_barrier()
        off = pl.multiple_of(s * _RPS, 8)
        pltpu.sync_copy(acc.at[pl.ds(off, _RPS)], out_hbm.at[pl.ds(off, _RPS)])

        @pl.when(s == _NS - 1)
        def _otail():
            pltpu.sync_copy(acc.at[pl.ds(_NS * _RPS, 16)],
                            out_hbm.at[pl.ds(_NS * _RPS, 16)])


def _pool(hm, seg):
    mesh = plsc.VectorSubcoreMesh(core_axis_name="c", subcore_axis_name="s")
    f = pl.kernel(
        _pool_body,
        out_type=jax.ShapeDtypeStruct((_S, _D), jnp.float32),
        mesh=mesh,
        scratch_types=[
            pltpu.VMEM((1, _PK), jnp.int32),
            pltpu.VMEM((_PK, _D), jnp.float32),
            pltpu.VMEM((104, _D), jnp.float32),
            pltpu.VMEM_SHARED((_S, _D), jnp.float32),
        ],
    )
    return f(hm, seg)


def kernel(x, edge_index, edge_attr, edge_weight, node_mask, subgraphs2nodes,
           be1_W, be1_b, be2_W, be2_b, nn1_W, nn1_b, nn2_W, nn2_b,
           bn_g, bn_b, eps):
    src = edge_index[0]
    dst2 = edge_index[1]
    mask2d = node_mask.reshape(_N, 1)

    eemb = _edge_mlp(edge_attr, be1_W, be1_b, be2_W, be2_b)

    h = x
    for l in range(_L):
        parts = _sc_layer(h, eemb, src, dst2, edge_weight, l)
        h = _node_update(h, parts,
                         nn1_W[l], nn1_b[l:l + 1], nn2_W[l], nn2_b[l:l + 1],
                         bn_g[l:l + 1], bn_b[l:l + 1],
                         eps[l].reshape(1, 1), mask2d,
                         residual=(l > 0), final_mask=(l == _L - 1))

    return _pool(h, subgraphs2nodes)
